# baseline (speedup 1.0000x reference)
import functools

import jax
import jax.numpy as jnp
from jax import lax
from jax.experimental import pallas as pl
from jax.experimental.pallas import tpu as pltpu
from jax.experimental.pallas import tpu_sc as plsc

_B, _NCAT, _NCONT, _V, _D = 16384, 26, 13, 100000, 32

_NROWS = _NCAT * _D
_NW = 32
_HROWS = _NROWS // 2
_ROWS_PER_W = _HROWS // _NW
_CHUNK = 4096
_NCHUNK = _B // _CHUNK


def _make_gather_body(off):
    def _gather_body(idx_hbm, tab_hbm, out_hbm, idx_v, row_v, out_a, out_b,
                     sem_a, sem_b):
        wid = lax.axis_index("s") * 2 + lax.axis_index("c")
        start = wid * _ROWS_PER_W
        end = start + _ROWS_PER_W
        gstart = off + start
        mid = jnp.minimum((lax.div(gstart, _D) + 1) * _D - off, end)
        bufs = (out_a, out_b)
        sems = (sem_a, sem_b)

        def store_chunk(r, c, buf, sem):
            pltpu.async_copy(buf, out_hbm.at[r, pl.ds(c * _CHUNK, _CHUNK)],
                             sem)

        def wait_store(buf, sem):
            pltpu.make_async_copy(buf, out_hbm.at[0, pl.ds(0, _CHUNK)],
                                  sem).wait()

        def process_row(r, carry):
            pltpu.sync_copy(tab_hbm.at[off + r], row_v)
            for c in range(_NCHUNK):
                buf, sem = bufs[c % 2], sems[c % 2]
                wait_store(buf, sem)

                @plsc.parallel_loop(0, _CHUNK, step=16, unroll=16)
                def _g(k):
                    i16 = idx_v[pl.ds(c * _CHUNK + k, 16)]
                    buf[pl.ds(k, 16)] = plsc.load_gather(row_v, [i16])

                store_chunk(r, c, buf, sem)
            return carry

        pltpu.sync_copy(idx_hbm.at[lax.div(gstart, _D)], idx_v)
        store_chunk(end - 1, 2, out_a, sem_a)
        store_chunk(end - 1, 3, out_b, sem_b)
        lax.fori_loop(start, mid, process_row, 0)

        @pl.when(mid < end)
        def _second_field():
            pltpu.sync_copy(idx_hbm.at[lax.div(off + mid, _D)], idx_v)

        lax.fori_loop(mid, end, process_row, 0)
        wait_store(out_a, sem_a)
        wait_store(out_b, sem_b)

    return _gather_body


def _sc_gather(idxT, tabT, off):
    mesh = plsc.VectorSubcoreMesh(core_axis_name="c", subcore_axis_name="s")
    k = pl.kernel(
        _make_gather_body(off),
        mesh=mesh,
        out_type=jax.ShapeDtypeStruct((_HROWS, _B), jnp.float32),
        scratch_types=[
            pltpu.VMEM((_B,), jnp.int32),
            pltpu.VMEM((_V,), jnp.float32),
            pltpu.VMEM((_CHUNK,), jnp.float32),
            pltpu.VMEM((_CHUNK,), jnp.float32),
            pltpu.SemaphoreType.DMA,
            pltpu.SemaphoreType.DMA,
        ],
        compiler_params=pltpu.CompilerParams(needs_layout_passes=False),
    )
    return k(idxT, tabT)


_BT = 2048


def _mlp1_body(cat_ref, x_ref, w1a_ref, w1c_ref, b1_ref, out_ref):
    x = x_ref[...]
    xc = jnp.where(jnp.isnan(x), 0.0, x)
    x1 = jnp.dot(w1a_ref[...], cat_ref[...], preferred_element_type=jnp.float32)
    x1 = x1 + jnp.dot(w1c_ref[...], xc, preferred_element_type=jnp.float32)
    out_ref[...] = x1 + b1_ref[...]


def _mlp1(cat1, xT, w1a1, w1c, b1):
    h1 = w1a1.shape[0]
    return pl.pallas_call(
        _mlp1_body,
        grid=(_B // _BT,),
        in_specs=[
            pl.BlockSpec((_HROWS, _BT), lambda i: (0, i)),
            pl.BlockSpec((_NCONT, _BT), lambda i: (0, i)),
            pl.BlockSpec((h1, _HROWS), lambda i: (0, 0)),
            pl.BlockSpec((h1, _NCONT), lambda i: (0, 0)),
            pl.BlockSpec((h1, 1), lambda i: (0, 0)),
        ],
        out_specs=pl.BlockSpec((h1, _BT), lambda i: (0, i)),
        out_shape=jax.ShapeDtypeStruct((h1, _B), jnp.float32),
    )(cat1, xT, w1a1, w1c, b1)


def _mlp2_body(acc_ref, cat_ref, w1a_ref, w2_ref, b2_ref, w3_ref, b3_ref,
               out_ref):
    x1 = acc_ref[...] + jnp.dot(w1a_ref[...], cat_ref[...],
                                preferred_element_type=jnp.float32)
    x1 = jnp.maximum(x1, 0.0)
    x2 = jnp.maximum(
        jnp.dot(w2_ref[...], x1, preferred_element_type=jnp.float32)
        + b2_ref[...], 0.0)
    out_ref[...] = (
        jnp.dot(w3_ref[...], x2, preferred_element_type=jnp.float32)
        + b3_ref[...])


def _mlp2(acc, cat2, w1a2, w2, b2, w3, b3):
    h1, h2 = w1a2.shape[0], w2.shape[0]
    return pl.pallas_call(
        _mlp2_body,
        grid=(_B // _BT,),
        in_specs=[
            pl.BlockSpec((h1, _BT), lambda i: (0, i)),
            pl.BlockSpec((_HROWS, _BT), lambda i: (0, i)),
            pl.BlockSpec((h1, _HROWS), lambda i: (0, 0)),
            pl.BlockSpec((h2, h1), lambda i: (0, 0)),
            pl.BlockSpec((h2, 1), lambda i: (0, 0)),
            pl.BlockSpec((1, h2), lambda i: (0, 0)),
            pl.BlockSpec((1, 1), lambda i: (0, 0)),
        ],
        out_specs=pl.BlockSpec((1, _BT), lambda i: (0, i)),
        out_shape=jax.ShapeDtypeStruct((1, _B), jnp.float32),
    )(acc, cat2, w1a2, w2, b2, w3, b3)


def kernel(x_cat, x_cont, cat_tables, cont_tables, W1, b1, W2, b2, Wout, bout):
    idxT = x_cat.T
    tabT = cat_tables.transpose(0, 2, 1).reshape(_NROWS, _V)
    cat1 = _sc_gather(idxT, tabT, 0)
    cat2 = _sc_gather(idxT, tabT, _HROWS)

    xT = x_cont.T
    w1c = jnp.einsum("id,jid->ji", cont_tables[:, 1, :],
                     W1[:, _NROWS:].reshape(-1, _NCONT, _D))
    acc = _mlp1(cat1, xT, W1[:, :_HROWS], w1c, b1.reshape(-1, 1))
    out = _mlp2(acc, cat2, W1[:, _HROWS:_NROWS], W2, b2.reshape(-1, 1),
                Wout, bout.reshape(-1, 1))
    return out.reshape(_B, 1)


# speedup vs baseline: 1.0543x; 1.0543x over previous
import functools

import jax
import jax.numpy as jnp
from jax import lax
from jax.experimental import pallas as pl
from jax.experimental.pallas import tpu as pltpu
from jax.experimental.pallas import tpu_sc as plsc

_B, _NCAT, _NCONT, _V, _D = 16384, 26, 13, 100000, 32

_NROWS = _NCAT * _D
_NW = 32
_ROWS_PER_W = _NROWS // _NW
_CHUNK = 4096
_NCHUNK = _B // _CHUNK


def _gather_body(idx_hbm, tab_hbm, out_hbm, idx_v, row_v, out_a, out_b,
                 sem_a, sem_b):
    wid = lax.axis_index("s") * 2 + lax.axis_index("c")
    start = wid * _ROWS_PER_W
    end = start + _ROWS_PER_W
    mid = jnp.minimum((lax.div(start, _D) + 1) * _D, end)
    bufs = (out_a, out_b)
    sems = (sem_a, sem_b)

    def store_chunk(r, c, buf, sem):
        pltpu.async_copy(buf, out_hbm.at[r, pl.ds(c * _CHUNK, _CHUNK)], sem)

    def wait_store(buf, sem):
        pltpu.make_async_copy(buf, out_hbm.at[0, pl.ds(0, _CHUNK)],
                              sem).wait()

    def process_row(r, carry):
        pltpu.sync_copy(tab_hbm.at[r], row_v)
        for c in range(_NCHUNK):
            buf, sem = bufs[c % 2], sems[c % 2]
            wait_store(buf, sem)

            @plsc.parallel_loop(0, _CHUNK, step=16, unroll=16)
            def _g(k):
                i16 = idx_v[pl.ds(c * _CHUNK + k, 16)]
                buf[pl.ds(k, 16)] = plsc.load_gather(row_v, [i16])

            store_chunk(r, c, buf, sem)
        return carry

    pltpu.sync_copy(idx_hbm.at[lax.div(start, _D)], idx_v)
    store_chunk(end - 1, 2, out_a, sem_a)
    store_chunk(end - 1, 3, out_b, sem_b)
    lax.fori_loop(start, mid, process_row, 0)

    @pl.when(mid < end)
    def _second_field():
        pltpu.sync_copy(idx_hbm.at[lax.div(mid, _D)], idx_v)

    lax.fori_loop(mid, end, process_row, 0)
    wait_store(out_a, sem_a)
    wait_store(out_b, sem_b)


def _sc_gather(idxT, tabT):
    mesh = plsc.VectorSubcoreMesh(core_axis_name="c", subcore_axis_name="s")
    k = pl.kernel(
        _gather_body,
        mesh=mesh,
        out_type=jax.ShapeDtypeStruct((_NROWS, _B), jnp.float32),
        scratch_types=[
            pltpu.VMEM((_B,), jnp.int32),
            pltpu.VMEM((_V,), jnp.float32),
            pltpu.VMEM((_CHUNK,), jnp.float32),
            pltpu.VMEM((_CHUNK,), jnp.float32),
            pltpu.SemaphoreType.DMA,
            pltpu.SemaphoreType.DMA,
        ],
        compiler_params=pltpu.CompilerParams(needs_layout_passes=False),
    )
    return k(idxT, tabT)


_BT = 4096


def _mlp_body(cat_ref, x_ref, w1a_ref, w1c_ref, b1_ref, w2_ref, b2_ref,
              w3_ref, b3_ref, out_ref):
    x = x_ref[...]
    xc = jnp.where(jnp.isnan(x), 0.0, x)
    x1 = jnp.dot(w1a_ref[...], cat_ref[...], preferred_element_type=jnp.float32)
    x1 = x1 + jnp.dot(w1c_ref[...], xc, preferred_element_type=jnp.float32)
    x1 = jnp.maximum(x1 + b1_ref[...], 0.0)
    x2 = jnp.maximum(
        jnp.dot(w2_ref[...], x1, preferred_element_type=jnp.float32)
        + b2_ref[...], 0.0)
    out_ref[...] = (
        jnp.dot(w3_ref[...], x2, preferred_element_type=jnp.float32)
        + b3_ref[...])


def _mlp(catT, xT, w1a, w1c, b1, w2, b2, w3, b3):
    h1, h2 = w1a.shape[0], w2.shape[0]
    return pl.pallas_call(
        _mlp_body,
        grid=(_B // _BT,),
        in_specs=[
            pl.BlockSpec((_NROWS, _BT), lambda i: (0, i)),
            pl.BlockSpec((_NCONT, _BT), lambda i: (0, i)),
            pl.BlockSpec((h1, _NROWS), lambda i: (0, 0)),
            pl.BlockSpec((h1, _NCONT), lambda i: (0, 0)),
            pl.BlockSpec((h1, 1), lambda i: (0, 0)),
            pl.BlockSpec((h2, h1), lambda i: (0, 0)),
            pl.BlockSpec((h2, 1), lambda i: (0, 0)),
            pl.BlockSpec((1, h2), lambda i: (0, 0)),
            pl.BlockSpec((1, 1), lambda i: (0, 0)),
        ],
        out_specs=pl.BlockSpec((1, _BT), lambda i: (0, i)),
        out_shape=jax.ShapeDtypeStruct((1, _B), jnp.float32),
    )(catT, xT, w1a, w1c, b1, w2, b2, w3, b3)


def kernel(x_cat, x_cont, cat_tables, cont_tables, W1, b1, W2, b2, Wout, bout):
    idxT = x_cat.T
    tabT = cat_tables.transpose(0, 2, 1).reshape(_NROWS, _V)
    catT = _sc_gather(idxT, tabT)

    xT = x_cont.T
    w1a = W1[:, :_NROWS]
    w1c = jnp.einsum("id,jid->ji", cont_tables[:, 1, :],
                     W1[:, _NROWS:].reshape(-1, _NCONT, _D))
    out = _mlp(catT, xT, w1a, w1c, b1.reshape(-1, 1), W2,
               b2.reshape(-1, 1), Wout, bout.reshape(-1, 1))
    return out.reshape(_B, 1)


# speedup vs baseline: 1.0620x; 1.0073x over previous
import functools

import jax
import jax.numpy as jnp
from jax import lax
from jax.experimental import pallas as pl
from jax.experimental.pallas import tpu as pltpu
from jax.experimental.pallas import tpu_sc as plsc

_B, _NCAT, _NCONT, _V, _D = 16384, 26, 13, 100000, 32

_NROWS = _NCAT * _D
_NW = 32
_ROWS_PER_W = _NROWS // _NW
_CHUNK = 4096
_NCHUNK = _B // _CHUNK


def _gather_body(idx_hbm, tab_hbm, out_hbm, idx_v, row_v, out_a, out_b,
                 sem_a, sem_b):
    wid = lax.axis_index("s") * 2 + lax.axis_index("c")
    start = wid * _ROWS_PER_W
    end = start + _ROWS_PER_W
    mid = jnp.minimum((lax.div(start, _D) + 1) * _D, end)
    bufs = (out_a, out_b)
    sems = (sem_a, sem_b)

    def store_chunk(r, c, buf, sem):
        pltpu.async_copy(buf, out_hbm.at[r, pl.ds(c * _CHUNK, _CHUNK)], sem)

    def wait_store(buf, sem):
        pltpu.make_async_copy(buf, out_hbm.at[0, pl.ds(0, _CHUNK)],
                              sem).wait()

    def process_row(r, carry):
        pltpu.sync_copy(tab_hbm.at[r], row_v)
        for c in range(_NCHUNK):
            buf, sem = bufs[c % 2], sems[c % 2]
            wait_store(buf, sem)

            @plsc.parallel_loop(0, _CHUNK, step=16, unroll=16)
            def _g(k):
                i16 = idx_v[pl.ds(c * _CHUNK + k, 16)]
                buf[pl.ds(k, 16)] = plsc.load_gather(row_v, [i16])

            store_chunk(r, c, buf, sem)
        return carry

    pltpu.sync_copy(idx_hbm.at[lax.div(start, _D)], idx_v)
    store_chunk(end - 1, 2, out_a, sem_a)
    store_chunk(end - 1, 3, out_b, sem_b)
    lax.fori_loop(start, mid, process_row, 0)

    @pl.when(mid < end)
    def _second_field():
        pltpu.sync_copy(idx_hbm.at[lax.div(mid, _D)], idx_v)

    lax.fori_loop(mid, end, process_row, 0)
    wait_store(out_a, sem_a)
    wait_store(out_b, sem_b)


def _sc_gather(idxT, tabT):
    mesh = plsc.VectorSubcoreMesh(core_axis_name="c", subcore_axis_name="s")
    k = pl.kernel(
        _gather_body,
        mesh=mesh,
        out_type=jax.ShapeDtypeStruct((_NROWS, _B), jnp.float32),
        scratch_types=[
            pltpu.VMEM((_B,), jnp.int32),
            pltpu.VMEM((_V,), jnp.float32),
            pltpu.VMEM((_CHUNK,), jnp.float32),
            pltpu.VMEM((_CHUNK,), jnp.float32),
            pltpu.SemaphoreType.DMA,
            pltpu.SemaphoreType.DMA,
        ],
        compiler_params=pltpu.CompilerParams(needs_layout_passes=False),
    )
    return k(idxT, tabT)


_BT = 2048


def _mlp_body(cat_ref, x_ref, w1a_ref, w1c_ref, b1_ref, w2_ref, b2_ref,
              w3_ref, b3_ref, out_ref):
    x = x_ref[...]
    xc = jnp.where(jnp.isnan(x), 0.0, x)
    x1 = jnp.dot(w1a_ref[...], cat_ref[...], preferred_element_type=jnp.float32)
    x1 = x1 + jnp.dot(w1c_ref[...], xc, preferred_element_type=jnp.float32)
    x1 = jnp.maximum(x1 + b1_ref[...], 0.0)
    x2 = jnp.maximum(
        jnp.dot(w2_ref[...], x1, preferred_element_type=jnp.float32)
        + b2_ref[...], 0.0)
    out_ref[...] = (
        jnp.dot(w3_ref[...], x2, preferred_element_type=jnp.float32)
        + b3_ref[...])


def _mlp(catT, xT, w1a, w1c, b1, w2, b2, w3, b3):
    h1, h2 = w1a.shape[0], w2.shape[0]
    return pl.pallas_call(
        _mlp_body,
        grid=(_B // _BT,),
        in_specs=[
            pl.BlockSpec((_NROWS, _BT), lambda i: (0, i)),
            pl.BlockSpec((_NCONT, _BT), lambda i: (0, i)),
            pl.BlockSpec((h1, _NROWS), lambda i: (0, 0)),
            pl.BlockSpec((h1, _NCONT), lambda i: (0, 0)),
            pl.BlockSpec((h1, 1), lambda i: (0, 0)),
            pl.BlockSpec((h2, h1), lambda i: (0, 0)),
            pl.BlockSpec((h2, 1), lambda i: (0, 0)),
            pl.BlockSpec((1, h2), lambda i: (0, 0)),
            pl.BlockSpec((1, 1), lambda i: (0, 0)),
        ],
        out_specs=pl.BlockSpec((1, _BT), lambda i: (0, i)),
        out_shape=jax.ShapeDtypeStruct((1, _B), jnp.float32),
    )(catT, xT, w1a, w1c, b1, w2, b2, w3, b3)


def kernel(x_cat, x_cont, cat_tables, cont_tables, W1, b1, W2, b2, Wout, bout):
    idxT = x_cat.T
    tabT = cat_tables.transpose(0, 2, 1).reshape(_NROWS, _V)
    catT = _sc_gather(idxT, tabT)

    xT = x_cont.T
    w1a = W1[:, :_NROWS]
    w1c = jnp.einsum("id,jid->ji", cont_tables[:, 1, :],
                     W1[:, _NROWS:].reshape(-1, _NCONT, _D))
    out = _mlp(catT, xT, w1a, w1c, b1.reshape(-1, 1), W2,
               b2.reshape(-1, 1), Wout, bout.reshape(-1, 1))
    return out.reshape(_B, 1)
